# baseline (speedup 1.0000x reference)
import functools

import jax
import jax.numpy as jnp
from jax import lax
from jax.experimental import pallas as pl
from jax.experimental.pallas import tpu as pltpu
from jax.experimental.pallas import tpu_sc as plsc

D = 64


def _project_body(eg_ref, ea_ref, eo_ref, ear_ref, w_ref, b_ref,
                  pg_ref, pa_ref, po_ref, par_ref):
    w = w_ref[...]
    b = b_ref[...]
    dims = (((1,), (1,)), ((), ()))
    f32 = jnp.float32
    pg_ref[...] = lax.dot_general(eg_ref[...], w[:, 0:D], dims,
                                  preferred_element_type=f32)
    pa_ref[...] = lax.dot_general(ea_ref[...], w[:, D:2 * D], dims,
                                  preferred_element_type=f32) + b
    po_ref[...] = lax.dot_general(eo_ref[...], w[:, 2 * D:3 * D], dims,
                                  preferred_element_type=f32)
    par_ref[...] = lax.dot_general(ear_ref[...], w[:, 3 * D:4 * D], dims,
                                   preferred_element_type=f32)


def _project(eg, ea, eo, ear, w, b):
    shapes = [jax.ShapeDtypeStruct((t.shape[0], D), jnp.float32)
              for t in (eg, ea, eo, ear)]
    return pl.pallas_call(_project_body, out_shape=shapes)(eg, ea, eo, ear, w, b)


@functools.cache
def _make_gather_sum(B, Vg, Va, Vo):
    info = plsc.get_sparse_core_info()
    NC, NS = info.num_cores, info.num_subcores
    NW = NC * NS
    C = B // NW
    G = C // 16
    mesh = plsc.VectorSubcoreMesh(core_axis_name="c", subcore_axis_name="s")

    @functools.partial(
        pl.kernel, mesh=mesh,
        out_type=jax.ShapeDtypeStruct((B, D), jnp.float32),
        compiler_params=pltpu.CompilerParams(use_tc_tiling_on_sc=False,
                                             needs_layout_passes=False),
        scratch_types=[
            pltpu.VMEM((C,), jnp.int32),
            pltpu.VMEM((C,), jnp.int32),
            pltpu.VMEM((C,), jnp.int32),
            pltpu.VMEM((C,), jnp.int32),
            pltpu.VMEM((Vg, D), jnp.float32),
            pltpu.VMEM((Va, D), jnp.float32),
            pltpu.VMEM((Vo, D), jnp.float32),
            pltpu.VMEM((C, D), jnp.float32),
            pltpu.SemaphoreType.DMA,
            pltpu.SemaphoreType.DMA,
        ],
    )
    def k(pg, pa, po, par, ig, ia, io, iar, out,
          igv, iav, iov, iarv, pgv, pav, pov, acc, sem, sem2):
        wid = lax.axis_index("s") * NC + lax.axis_index("c")
        base = wid * C
        pltpu.async_copy(iar.at[pl.ds(base, C)], iarv, sem2).wait()
        area_cp = pltpu.async_copy(par.at[iarv], acc, sem)
        d5 = [
            pltpu.async_copy(par.at[iarv], acc, sem),
            pltpu.async_copy(par.at[iarv], acc, sem),
            pltpu.async_copy(par.at[iarv], acc, sem),
        ]
        for cp in d5:
            cp.wait()
        stage = [
            pltpu.async_copy(ig.at[pl.ds(base, C)], igv, sem2),
            pltpu.async_copy(ia.at[pl.ds(base, C)], iav, sem2),
            pltpu.async_copy(io.at[pl.ds(base, C)], iov, sem2),
            pltpu.async_copy(pg, pgv, sem2),
            pltpu.async_copy(pa, pav, sem2),
            pltpu.async_copy(po, pov, sem2),
        ]
        for cp in stage:
            cp.wait()
        area_cp.wait()

        lane = lax.iota(jnp.int32, 16)

        def body(g, carry):
            s0 = g * 16
            sids = s0 + lane
            idxg = igv[pl.ds(s0, 16)]
            idxa = iav[pl.ds(s0, 16)]
            idxo = iov[pl.ds(s0, 16)]
            for c in range(D):
                cc = jnp.full((16,), c, jnp.int32)
                v = (plsc.load_gather(pgv, [idxg, cc])
                     + plsc.load_gather(pav, [idxa, cc])
                     + plsc.load_gather(pov, [idxo, cc]))
                plsc.addupdate_scatter(acc, [sids, cc], v)
            return carry

        lax.fori_loop(0, G, body, 0)
        pltpu.sync_copy(acc, out.at[pl.ds(base, C)])

    return k


def kernel(gender_idx, age_idx, occupation_idx, area_idx,
           emb_gender, emb_age, emb_occupation, emb_area, lin_w, lin_b):
    B = gender_idx.shape[0]
    gi = gender_idx.astype(jnp.int32)
    ai = age_idx.astype(jnp.int32)
    oi = occupation_idx.astype(jnp.int32)
    ari = area_idx.astype(jnp.int32)

    pg, pa, po, par = _project(emb_gender, emb_age, emb_occupation, emb_area,
                               lin_w, lin_b.reshape(1, D))
    return _make_gather_sum(B, pg.shape[0], pa.shape[0], po.shape[0])(
        pg, pa, po, par, gi, ai, oi, ari)


# speedup vs baseline: 1.0449x; 1.0449x over previous
import functools

import jax
import jax.numpy as jnp
from jax import lax
from jax.experimental import pallas as pl
from jax.experimental.pallas import tpu as pltpu
from jax.experimental.pallas import tpu_sc as plsc

D = 64


def _project_body(eg_ref, ea_ref, eo_ref, ear_ref, w_ref, b_ref,
                  pg_ref, pa_ref, po_ref, par_ref):
    w = w_ref[...]
    b = b_ref[...]
    dims = (((1,), (1,)), ((), ()))
    f32 = jnp.float32
    pg_ref[...] = lax.dot_general(eg_ref[...], w[:, 0:D], dims,
                                  preferred_element_type=f32)
    pa_ref[...] = lax.dot_general(ea_ref[...], w[:, D:2 * D], dims,
                                  preferred_element_type=f32) + b
    po_ref[...] = lax.dot_general(eo_ref[...], w[:, 2 * D:3 * D], dims,
                                  preferred_element_type=f32)
    par_ref[...] = lax.dot_general(ear_ref[...], w[:, 3 * D:4 * D], dims,
                                   preferred_element_type=f32)


def _project(eg, ea, eo, ear, w, b):
    shapes = [jax.ShapeDtypeStruct((t.shape[0], D), jnp.float32)
              for t in (eg, ea, eo, ear)]
    return pl.pallas_call(_project_body, out_shape=shapes)(eg, ea, eo, ear, w, b)


@functools.cache
def _make_gather_sum(B, Vg, Va, Vo):
    info = plsc.get_sparse_core_info()
    NC, NS = info.num_cores, info.num_subcores
    NW = NC * NS
    C = B // NW
    G = C // 16
    mesh = plsc.VectorSubcoreMesh(core_axis_name="c", subcore_axis_name="s")

    @functools.partial(
        pl.kernel, mesh=mesh,
        out_type=jax.ShapeDtypeStruct((B, D), jnp.float32),
        compiler_params=pltpu.CompilerParams(use_tc_tiling_on_sc=False,
                                             needs_layout_passes=False),
        scratch_types=[
            pltpu.VMEM((C,), jnp.int32),
            pltpu.VMEM((C,), jnp.int32),
            pltpu.VMEM((C,), jnp.int32),
            pltpu.VMEM((C,), jnp.int32),
            pltpu.VMEM((Vg, D), jnp.float32),
            pltpu.VMEM((Va, D), jnp.float32),
            pltpu.VMEM((Vo, D), jnp.float32),
            pltpu.VMEM((C, D), jnp.float32),
            pltpu.SemaphoreType.DMA,
            pltpu.SemaphoreType.DMA,
        ],
    )
    def k(pg, pa, po, par, ig, ia, io, iar, out,
          igv, iav, iov, iarv, pgv, pav, pov, acc, sem, sem2):
        wid = lax.axis_index("s") * NC + lax.axis_index("c")
        base = wid * C
        pltpu.async_copy(iar.at[pl.ds(base, C)], iarv, sem2).wait()
        area_cp = pltpu.async_copy(par.at[iarv], acc, sem)
        stage = [
            pltpu.async_copy(ig.at[pl.ds(base, C)], igv, sem2),
            pltpu.async_copy(ia.at[pl.ds(base, C)], iav, sem2),
            pltpu.async_copy(io.at[pl.ds(base, C)], iov, sem2),
            pltpu.async_copy(pg, pgv, sem2),
            pltpu.async_copy(pa, pav, sem2),
            pltpu.async_copy(po, pov, sem2),
        ]
        for cp in stage:
            cp.wait()
        area_cp.wait()

        lane = lax.iota(jnp.int32, 16)

        def body(g, carry):
            s0 = g * 16
            sids = s0 + lane
            idxg = igv[pl.ds(s0, 16)]
            idxa = iav[pl.ds(s0, 16)]
            idxo = iov[pl.ds(s0, 16)]
            for c in range(D):
                cc = jnp.full((16,), c, jnp.int32)
                v = (plsc.load_gather(pgv, [idxg, cc])
                     + plsc.load_gather(pav, [idxa, cc])
                     + plsc.load_gather(pov, [idxo, cc]))
                plsc.addupdate_scatter(acc, [sids, cc], v)
            return carry

        lax.fori_loop(0, G, body, 0)
        pltpu.sync_copy(acc, out.at[pl.ds(base, C)])

    return k


def kernel(gender_idx, age_idx, occupation_idx, area_idx,
           emb_gender, emb_age, emb_occupation, emb_area, lin_w, lin_b):
    B = gender_idx.shape[0]
    gi = gender_idx.astype(jnp.int32)
    ai = age_idx.astype(jnp.int32)
    oi = occupation_idx.astype(jnp.int32)
    ari = area_idx.astype(jnp.int32)

    pg, pa, po, par = _project(emb_gender, emb_age, emb_occupation, emb_area,
                               lin_w, lin_b.reshape(1, D))
    return _make_gather_sum(B, pg.shape[0], pa.shape[0], po.shape[0])(
        pg, pa, po, par, gi, ai, oi, ari)


# speedup vs baseline: 2.1171x; 2.0261x over previous
import functools

import jax
import jax.numpy as jnp
from jax import lax
from jax.experimental import pallas as pl
from jax.experimental.pallas import tpu as pltpu
from jax.experimental.pallas import tpu_sc as plsc

D = 64
SB = 512


@functools.cache
def _make_area_gather(B, V):
    info = plsc.get_sparse_core_info()
    NC, NS = info.num_cores, info.num_subcores
    NW = NC * NS
    C = B // NW
    mesh = plsc.VectorSubcoreMesh(core_axis_name="c", subcore_axis_name="s")

    @functools.partial(
        pl.kernel, mesh=mesh,
        out_type=jax.ShapeDtypeStruct((B, D), jnp.float32),
        compiler_params=pltpu.CompilerParams(use_tc_tiling_on_sc=False,
                                             needs_layout_passes=False),
        scratch_types=[
            pltpu.VMEM((C,), jnp.int32),
            pltpu.VMEM((C, D), jnp.float32),
            pltpu.SemaphoreType.DMA,
        ],
    )
    def k(tab, idx, out, idxv, rows, sem):
        wid = lax.axis_index("s") * NC + lax.axis_index("c")
        base = wid * C
        pltpu.sync_copy(idx.at[pl.ds(base, C)], idxv)
        pltpu.async_copy(tab.at[idxv], rows, sem).wait()
        pltpu.sync_copy(rows, out.at[pl.ds(base, C)])

    return k


def _combine_body(gi_ref, ai_ref, oi_ref, eg_ref, ea_ref, eo_ref, ar_ref,
                  w_ref, b_ref, out_ref):
    f32 = jnp.float32
    dimsT = (((0,), (0,)), ((), ()))

    def lookup(idx_ref, tab_ref):
        v = tab_ref.shape[0]
        iot = lax.broadcasted_iota(jnp.int32, (v, SB), 0)
        onehot_t = (idx_ref[0] == iot).astype(f32)
        return lax.dot_general(onehot_t, tab_ref[...], dimsT,
                               preferred_element_type=f32)

    x = jnp.concatenate([
        lookup(gi_ref, eg_ref),
        lookup(ai_ref, ea_ref),
        lookup(oi_ref, eo_ref),
        ar_ref[...],
    ], axis=1)
    out_ref[...] = lax.dot_general(x, w_ref[...], (((1,), (1,)), ((), ())),
                                   preferred_element_type=f32) + b_ref[...]


def _combine(gi3, ai3, oi3, eg, ea, eo, ar_rows, w, b2):
    nb = gi3.shape[0]
    full = lambda s: pl.BlockSpec(s, lambda i: (0,) * len(s))
    return pl.pallas_call(
        _combine_body,
        grid=(nb,),
        in_specs=[
            pl.BlockSpec((1, 1, SB), lambda i: (i, 0, 0)),
            pl.BlockSpec((1, 1, SB), lambda i: (i, 0, 0)),
            pl.BlockSpec((1, 1, SB), lambda i: (i, 0, 0)),
            full(eg.shape),
            full(ea.shape),
            full(eo.shape),
            pl.BlockSpec((SB, D), lambda i: (i, 0)),
            full(w.shape),
            full(b2.shape),
        ],
        out_specs=pl.BlockSpec((SB, D), lambda i: (i, 0)),
        out_shape=jax.ShapeDtypeStruct((nb * SB, D), jnp.float32),
    )(gi3, ai3, oi3, eg, ea, eo, ar_rows, w, b2)


def kernel(gender_idx, age_idx, occupation_idx, area_idx,
           emb_gender, emb_age, emb_occupation, emb_area, lin_w, lin_b):
    B = gender_idx.shape[0]
    nb = B // SB
    gi3 = gender_idx.astype(jnp.int32).reshape(nb, 1, SB)
    ai3 = age_idx.astype(jnp.int32).reshape(nb, 1, SB)
    oi3 = occupation_idx.astype(jnp.int32).reshape(nb, 1, SB)
    ari = area_idx.astype(jnp.int32)

    ar_rows = _make_area_gather(B, emb_area.shape[0])(emb_area, ari)
    return _combine(gi3, ai3, oi3, emb_gender, emb_age, emb_occupation,
                    ar_rows, lin_w, lin_b.reshape(1, D))


# speedup vs baseline: 2.9092x; 1.3741x over previous
import functools

import jax
import jax.numpy as jnp
from jax import lax
from jax.experimental import pallas as pl
from jax.experimental.pallas import tpu as pltpu
from jax.experimental.pallas import tpu_sc as plsc

D = 64
SB = 512


@functools.cache
def _make_area_gather(B, V):
    info = plsc.get_sparse_core_info()
    NC, NS = info.num_cores, info.num_subcores
    NW = NC * NS
    C = B // NW
    mesh = plsc.VectorSubcoreMesh(core_axis_name="c", subcore_axis_name="s")

    @functools.partial(
        pl.kernel, mesh=mesh,
        out_type=jax.ShapeDtypeStruct((B, D), jnp.float32),
        compiler_params=pltpu.CompilerParams(use_tc_tiling_on_sc=False,
                                             needs_layout_passes=False),
        scratch_types=[
            pltpu.VMEM((C,), jnp.int32),
            pltpu.VMEM((C, D), jnp.float32),
            pltpu.SemaphoreType.DMA,
        ],
    )
    def k(tab, idx, out, idxv, rows, sem):
        wid = lax.axis_index("s") * NC + lax.axis_index("c")
        base = wid * C
        pltpu.sync_copy(idx.at[pl.ds(base, C)], idxv)
        pltpu.async_copy(tab.at[idxv], rows, sem).wait()
        pltpu.sync_copy(rows, out.at[pl.ds(base, C)])

    return k


def _combine_body(gi_ref, ai_ref, oi_ref, eg_ref, ea_ref, eo_ref, ar_ref,
                  w_ref, b_ref, out_ref):
    f32 = jnp.float32
    B = gi_ref.shape[1]
    w = w_ref[...]
    dims_t = (((0,), (0,)), ((), ()))
    dims_k = (((1,), (1,)), ((), ()))

    pcat = jnp.concatenate([
        lax.dot_general(eg_ref[...], w[:, 0:D], dims_k,
                        preferred_element_type=f32),
        lax.dot_general(ea_ref[...], w[:, D:2 * D], dims_k,
                        preferred_element_type=f32),
        lax.dot_general(eo_ref[...], w[:, 2 * D:3 * D], dims_k,
                        preferred_element_type=f32),
    ], axis=0)

    def oh(idx_ref, v):
        iot = lax.broadcasted_iota(jnp.int32, (v, B), 0)
        return (idx_ref[...] == iot).astype(f32)

    ohcat = jnp.concatenate([
        oh(gi_ref, eg_ref.shape[0]),
        oh(ai_ref, ea_ref.shape[0]),
        oh(oi_ref, eo_ref.shape[0]),
    ], axis=0)

    out_ref[...] = (
        lax.dot_general(ohcat, pcat, dims_t, preferred_element_type=f32)
        + lax.dot_general(ar_ref[...], w[:, 3 * D:4 * D], dims_k,
                          preferred_element_type=f32)
        + b_ref[...]
    )


def _combine(gi2, ai2, oi2, eg, ea, eo, ar_rows, w, b2):
    B = gi2.shape[1]
    return pl.pallas_call(
        _combine_body,
        out_shape=jax.ShapeDtypeStruct((B, D), jnp.float32),
    )(gi2, ai2, oi2, eg, ea, eo, ar_rows, w, b2)


def kernel(gender_idx, age_idx, occupation_idx, area_idx,
           emb_gender, emb_age, emb_occupation, emb_area, lin_w, lin_b):
    B = gender_idx.shape[0]
    gi2 = gender_idx.astype(jnp.int32).reshape(1, B)
    ai2 = age_idx.astype(jnp.int32).reshape(1, B)
    oi2 = occupation_idx.astype(jnp.int32).reshape(1, B)
    ari = area_idx.astype(jnp.int32)

    ar_rows = _make_area_gather(B, emb_area.shape[0])(emb_area, ari)
    return _combine(gi2, ai2, oi2, emb_gender, emb_age, emb_occupation,
                    ar_rows, lin_w, lin_b.reshape(1, D))


# speedup vs baseline: 2.9192x; 1.0034x over previous
import functools

import jax
import jax.numpy as jnp
from jax import lax
from jax.experimental import pallas as pl
from jax.experimental.pallas import tpu as pltpu
from jax.experimental.pallas import tpu_sc as plsc

D = 64


@functools.cache
def _make_area_gather(B):
    info = plsc.get_sparse_core_info()
    NC, NS = info.num_cores, info.num_subcores
    NW = NC * NS
    C = B // NW
    mesh = plsc.VectorSubcoreMesh(core_axis_name="c", subcore_axis_name="s")

    @functools.partial(
        pl.kernel, mesh=mesh,
        out_type=jax.ShapeDtypeStruct((B, D), jnp.float32),
        compiler_params=pltpu.CompilerParams(use_tc_tiling_on_sc=False,
                                             needs_layout_passes=False),
        scratch_types=[
            pltpu.VMEM((C,), jnp.int32),
            pltpu.VMEM((C, D), jnp.float32),
            pltpu.SemaphoreType.DMA,
        ],
    )
    def k(tab, idx, out, idxv, rows, sem):
        wid = lax.axis_index("s") * NC + lax.axis_index("c")
        base = wid * C
        pltpu.sync_copy(idx.at[pl.ds(base, C)], idxv)
        pltpu.async_copy(tab.at[idxv], rows, sem).wait()
        pltpu.sync_copy(rows, out.at[pl.ds(base, C)])

    return k


def _combine_body(gi_ref, ai_ref, oi_ref, eg_ref, ea_ref, eo_ref, ar_ref,
                  w_ref, b_ref, out_ref):
    f32 = jnp.float32
    B = gi_ref.shape[1]
    w = w_ref[...]
    dims_t = (((0,), (0,)), ((), ()))
    dims_k = (((1,), (1,)), ((), ()))

    pcat = jnp.concatenate([
        lax.dot_general(eg_ref[...], w[:, 0:D], dims_k,
                        preferred_element_type=f32),
        lax.dot_general(ea_ref[...], w[:, D:2 * D], dims_k,
                        preferred_element_type=f32),
        lax.dot_general(eo_ref[...], w[:, 2 * D:3 * D], dims_k,
                        preferred_element_type=f32),
    ], axis=0)

    def oh(idx_ref, v):
        iot = lax.broadcasted_iota(jnp.int32, (v, B), 0)
        return (idx_ref[...] == iot).astype(f32)

    ohcat = jnp.concatenate([
        oh(gi_ref, eg_ref.shape[0]),
        oh(ai_ref, ea_ref.shape[0]),
        oh(oi_ref, eo_ref.shape[0]),
    ], axis=0)

    out_ref[...] = (
        lax.dot_general(ohcat, pcat, dims_t, preferred_element_type=f32)
        + lax.dot_general(ar_ref[...], w[:, 3 * D:4 * D], dims_k,
                          preferred_element_type=f32)
        + b_ref[...]
    )


def _combine(gi2, ai2, oi2, eg, ea, eo, ar_rows, w, b2):
    B = gi2.shape[1]
    return pl.pallas_call(
        _combine_body,
        out_shape=jax.ShapeDtypeStruct((B, D), jnp.float32),
    )(gi2, ai2, oi2, eg, ea, eo, ar_rows, w, b2)


def kernel(gender_idx, age_idx, occupation_idx, area_idx,
           emb_gender, emb_age, emb_occupation, emb_area, lin_w, lin_b):
    B = gender_idx.shape[0]
    gi2 = gender_idx.astype(jnp.int32).reshape(1, B)
    ai2 = age_idx.astype(jnp.int32).reshape(1, B)
    oi2 = occupation_idx.astype(jnp.int32).reshape(1, B)
    ari = area_idx.astype(jnp.int32)

    ar_rows = _make_area_gather(B)(emb_area, ari)
    return _combine(gi2, ai2, oi2, emb_gender, emb_age, emb_occupation,
                    ar_rows, lin_w, lin_b.reshape(1, D))
